# baseline (speedup 1.0000x reference)
import functools

import jax
import jax.numpy as jnp
from jax import lax
from jax.experimental import pallas as pl
from jax.experimental.pallas import tpu as pltpu
from jax.experimental.pallas import tpu_sc as plsc

N = 10000
NP = 10240
E = 320000
EPS = 1e-5

NC = 2
NS = 16
EB = 128
E2 = 327680
NB2 = E2 // EB // (NC * NS)
CH = 16
RPT = NP // NS
RB = 128

_MESH = plsc.VectorSubcoreMesh(core_axis_name="c", subcore_axis_name="s")


def _deg_hist(dst, ones_blk, zeros_blk):
    @functools.partial(
        pl.kernel,
        out_type=jax.ShapeDtypeStruct((NC, NP, 128), jnp.float32),
        mesh=_MESH,
        scratch_types=[
            pltpu.VMEM((CH, EB), jnp.int32),
            pltpu.VMEM((EB, 128), jnp.float32),
            pltpu.VMEM((RB, 128), jnp.float32),
            pltpu.VMEM_SHARED((NP, 128), jnp.float32),
            pltpu.SemaphoreType.DMA,
            pltpu.SemaphoreType.DMA,
        ],
    )
    def deg_kernel(dst_hbm, ones_hbm, zeros_hbm, out_hbm, didx, ones_v,
                   stage, acc, sem0, sem1):
        c = lax.axis_index("c")
        s = lax.axis_index("s")
        pltpu.sync_copy(zeros_hbm, stage)

        @pl.loop(0, RPT // RB)
        def _(k):
            pltpu.sync_copy(stage, acc.at[pl.ds(s * RPT + k * RB, RB)])

        pltpu.sync_copy(ones_hbm, ones_v)
        b0 = (c * NS + s) * NB2
        plsc.subcore_barrier()

        def fire(i, sem):
            pltpu.async_copy(ones_v, acc.at[didx.at[i]], sem, add=True)

        def drain(i, sem):
            pltpu.make_async_copy(ones_v, acc.at[didx.at[i]], sem).wait()

        @pl.loop(0, NB2 // CH)
        def _(ci):
            pltpu.sync_copy(dst_hbm.at[pl.ds(b0 + ci * CH, CH)], didx)
            fire(0, sem0)
            fire(1, sem1)

            @pl.loop(0, CH // 2 - 1)
            def _(k):
                i0 = 2 * k
                drain(i0, sem0)
                fire(i0 + 2, sem0)
                drain(i0 + 1, sem1)
                fire(i0 + 3, sem1)

            drain(CH - 2, sem0)
            drain(CH - 1, sem1)

        plsc.subcore_barrier()

        @pl.loop(0, RPT // RB)
        def _(k):
            r0 = s * RPT + k * RB
            pltpu.sync_copy(acc.at[pl.ds(r0, RB)], stage)
            pltpu.sync_copy(stage, out_hbm.at[c].at[pl.ds(r0, RB)])

    return deg_kernel(dst, ones_blk, zeros_blk)


def _segment_sum(src, dst, table, zeros_blk, width):
    @functools.partial(
        pl.kernel,
        out_type=jax.ShapeDtypeStruct((NC, NP, width), jnp.float32),
        mesh=_MESH,
        scratch_types=[
            pltpu.VMEM((CH, EB), jnp.int32),
            pltpu.VMEM((CH, EB), jnp.int32),
            pltpu.VMEM((EB, width), jnp.float32),
            pltpu.VMEM((EB, width), jnp.float32),
            pltpu.VMEM_SHARED((NP, width), jnp.float32),
            pltpu.SemaphoreType.DMA,
            pltpu.SemaphoreType.DMA,
            pltpu.SemaphoreType.DMA,
            pltpu.SemaphoreType.DMA,
        ],
    )
    def seg_kernel(src_hbm, dst_hbm, tab_hbm, zeros_hbm, out_hbm,
                   sidx, didx, rows0, rows1, acc, gs0, gs1, ss0, ss1):
        c = lax.axis_index("c")
        s = lax.axis_index("s")
        pltpu.sync_copy(zeros_hbm, rows0)

        @pl.loop(0, RPT // RB)
        def _(k):
            pltpu.sync_copy(rows0, acc.at[pl.ds(s * RPT + k * RB, RB)])

        b0 = (c * NS + s) * NB2
        plsc.subcore_barrier()

        def fireg(i, rows, sem):
            pltpu.async_copy(tab_hbm.at[sidx.at[i]], rows, sem)

        def draing(rows, sem):
            pltpu.make_async_copy(tab_hbm.at[sidx.at[0]], rows, sem).wait()

        def fires(i, rows, sem):
            pltpu.async_copy(rows, acc.at[didx.at[i]], sem, add=True)

        def drains(i, rows, sem):
            pltpu.make_async_copy(rows, acc.at[didx.at[i]], sem).wait()

        @pl.loop(0, NB2 // CH)
        def _(ci):
            pltpu.sync_copy(src_hbm.at[pl.ds(b0 + ci * CH, CH)], sidx)
            pltpu.sync_copy(dst_hbm.at[pl.ds(b0 + ci * CH, CH)], didx)

            fireg(0, rows0, gs0)
            draing(rows0, gs0)
            fires(0, rows0, ss0)
            fireg(1, rows1, gs1)
            draing(rows1, gs1)
            fires(1, rows1, ss1)
            drains(0, rows0, ss0)
            fireg(2, rows0, gs0)

            @pl.loop(1, CH // 2 - 1)
            def _(k):
                i = 2 * k
                draing(rows0, gs0)
                fires(i, rows0, ss0)
                drains(i - 1, rows1, ss1)
                fireg(i + 1, rows1, gs1)
                draing(rows1, gs1)
                fires(i + 1, rows1, ss1)
                drains(i, rows0, ss0)
                fireg(i + 2, rows0, gs0)

            draing(rows0, gs0)
            fires(CH - 2, rows0, ss0)
            drains(CH - 3, rows1, ss1)
            fireg(CH - 1, rows1, gs1)
            draing(rows1, gs1)
            fires(CH - 1, rows1, ss1)
            drains(CH - 2, rows0, ss0)
            drains(CH - 1, rows1, ss1)

        plsc.subcore_barrier()

        @pl.loop(0, RPT // RB)
        def _(k):
            r0 = s * RPT + k * RB
            pltpu.sync_copy(acc.at[pl.ds(r0, RB)], rows0)
            pltpu.sync_copy(rows0, out_hbm.at[c].at[pl.ds(r0, RB)])

    return seg_kernel(src, dst, table, zeros_blk)


BR = 1000


def _deg_specs():
    return [
        pl.BlockSpec((1, BR, 128), lambda i: (0, i, 0)),
        pl.BlockSpec((1, BR, 128), lambda i: (1, i, 0)),
    ]


def _dis_from(pa_ref, pb_ref):
    deg = 1.0 + pa_ref[0][:, :1] + pb_ref[0][:, :1]
    return lax.rsqrt(deg), 1.0 / deg


def _tc_first(x, W1, degp):
    d_out = W1.shape[1]

    def body(x_ref, w_ref, pa_ref, pb_ref, h_ref, hs_ref):
        dis, _ = _dis_from(pa_ref, pb_ref)
        h = jnp.dot(x_ref[...], w_ref[...], preferred_element_type=jnp.float32)
        h_ref[...] = h
        hs_ref[...] = h * dis

    return pl.pallas_call(
        body,
        grid=(N // BR,),
        in_specs=[
            pl.BlockSpec((BR, x.shape[1]), lambda i: (i, 0)),
            pl.BlockSpec(W1.shape, lambda i: (0, 0)),
        ] + _deg_specs(),
        out_specs=[
            pl.BlockSpec((BR, d_out), lambda i: (i, 0)),
            pl.BlockSpec((BR, d_out), lambda i: (i, 0)),
        ],
        out_shape=[
            jax.ShapeDtypeStruct((N, d_out), jnp.float32),
            jax.ShapeDtypeStruct((N, d_out), jnp.float32),
        ],
    )(x, W1, degp, degp)


def _tc_mid(S, Hp, degp, b, g, beta, Wn):
    w_in = Hp.shape[1]
    d_out = Wn.shape[1]

    def body(sa_ref, sb_ref, hp_ref, pa_ref, pb_ref, b_ref, g_ref, beta_ref,
             w_ref, h_ref, hs_ref):
        dis, dis2 = _dis_from(pa_ref, pb_ref)
        z = (dis * (sa_ref[0][:, :w_in] + sb_ref[0][:, :w_in])
             + dis2 * hp_ref[...] + b_ref[...])
        scale = g_ref[...] / jnp.sqrt(1.0 + EPS)
        xact = jnp.maximum(z * scale + beta_ref[...], 0.0)
        h = jnp.dot(xact, w_ref[...], preferred_element_type=jnp.float32)
        h_ref[...] = h
        if d_out == 128:
            hs_ref[...] = h * dis
        else:
            hs_ref[...] = jnp.concatenate(
                [h * dis, jnp.zeros((h.shape[0], 128 - d_out), jnp.float32)],
                axis=1)

    return pl.pallas_call(
        body,
        grid=(N // BR,),
        in_specs=[
            pl.BlockSpec((1, BR, 128), lambda i: (0, i, 0)),
            pl.BlockSpec((1, BR, 128), lambda i: (1, i, 0)),
            pl.BlockSpec((BR, w_in), lambda i: (i, 0)),
        ] + _deg_specs() + [
            pl.BlockSpec((1, w_in), lambda i: (0, 0)),
            pl.BlockSpec((1, w_in), lambda i: (0, 0)),
            pl.BlockSpec((1, w_in), lambda i: (0, 0)),
            pl.BlockSpec(Wn.shape, lambda i: (0, 0)),
        ],
        out_specs=[
            pl.BlockSpec((BR, d_out), lambda i: (i, 0)),
            pl.BlockSpec((BR, 128), lambda i: (i, 0)),
        ],
        out_shape=[
            jax.ShapeDtypeStruct((N, d_out), jnp.float32),
            jax.ShapeDtypeStruct((N, 128), jnp.float32),
        ],
    )(S, S, Hp, degp, degp, b, g, beta, Wn)


def _tc_final(S, Hp, degp, b, g, beta):
    w_in = Hp.shape[1]

    def body(sa_ref, sb_ref, hp_ref, pa_ref, pb_ref, b_ref, g_ref, beta_ref,
             o_ref):
        dis, dis2 = _dis_from(pa_ref, pb_ref)
        z = (dis * (sa_ref[0][:, :w_in] + sb_ref[0][:, :w_in])
             + dis2 * hp_ref[...] + b_ref[...])
        scale = g_ref[...] / jnp.sqrt(1.0 + EPS)
        o_ref[...] = z * scale + beta_ref[...]

    return pl.pallas_call(
        body,
        grid=(N // BR,),
        in_specs=[
            pl.BlockSpec((1, BR, 128), lambda i: (0, i, 0)),
            pl.BlockSpec((1, BR, 128), lambda i: (1, i, 0)),
            pl.BlockSpec((BR, w_in), lambda i: (i, 0)),
        ] + _deg_specs() + [
            pl.BlockSpec((1, w_in), lambda i: (0, 0)),
            pl.BlockSpec((1, w_in), lambda i: (0, 0)),
            pl.BlockSpec((1, w_in), lambda i: (0, 0)),
        ],
        out_specs=pl.BlockSpec((BR, w_in), lambda i: (i, 0)),
        out_shape=jax.ShapeDtypeStruct((N, w_in), jnp.float32),
    )(S, S, Hp, degp, degp, b, g, beta)


def kernel(x, edge_index, W1, b1, W2, b2, W3, b3,
           g1, beta1, g2, beta2, g3, beta3):
    npad = E2 - E
    pad_src = (jnp.arange(npad, dtype=jnp.int32) * 37) % N
    pad_dst = N + (jnp.arange(npad, dtype=jnp.int32) % (NP - N))
    src = jnp.concatenate([edge_index[0], pad_src]).reshape(E2 // EB, EB)
    dst = jnp.concatenate([edge_index[1], pad_dst]).reshape(E2 // EB, EB)

    ones_blk = jnp.ones((EB, 128), jnp.float32)
    zeros128 = jnp.zeros((RB, 128), jnp.float32)

    degp = _deg_hist(dst, ones_blk, zeros128)

    b1r, b2r, b3r = b1.reshape(1, -1), b2.reshape(1, -1), b3.reshape(1, -1)
    g1r, g2r, g3r = g1.reshape(1, -1), g2.reshape(1, -1), g3.reshape(1, -1)
    be1r, be2r, be3r = (beta1.reshape(1, -1), beta2.reshape(1, -1),
                        beta3.reshape(1, -1))

    H1, Hs1 = _tc_first(x, W1, degp)
    S1 = _segment_sum(src, dst, Hs1, zeros128, 128)
    H2, Hs2 = _tc_mid(S1, H1, degp, b1r, g1r, be1r, W2)
    S2 = _segment_sum(src, dst, Hs2, zeros128, 128)
    H3, Hs3 = _tc_mid(S2, H2, degp, b2r, g2r, be2r, W3)
    S3 = _segment_sum(src, dst, Hs3, zeros128, 128)
    out = _tc_final(S3, H3, degp, b3r, g3r, be3r)
    return out


# speedup vs baseline: 1.1090x; 1.1090x over previous
import functools

import jax
import jax.numpy as jnp
from jax import lax
from jax.experimental import pallas as pl
from jax.experimental.pallas import tpu as pltpu
from jax.experimental.pallas import tpu_sc as plsc

N = 10000
NP = 10240
E = 320000
EPS = 1e-5

NC = 2
NS = 16
EB = 128
E2 = 327680
NB2 = E2 // EB // (NC * NS)
CH = 16
RPT = NP // NS
RB = 128

_MESH = plsc.VectorSubcoreMesh(core_axis_name="c", subcore_axis_name="s")


def _deg_hist(dst, ones_blk, zeros_blk):
    @functools.partial(
        pl.kernel,
        out_type=jax.ShapeDtypeStruct((NC, NP, 128), jnp.float32),
        mesh=_MESH,
        scratch_types=[
            pltpu.VMEM((CH, EB), jnp.int32),
            pltpu.VMEM((EB, 128), jnp.float32),
            pltpu.VMEM((RB, 128), jnp.float32),
            pltpu.VMEM_SHARED((NP, 128), jnp.float32),
            pltpu.SemaphoreType.DMA,
            pltpu.SemaphoreType.DMA,
        ],
    )
    def deg_kernel(dst_hbm, ones_hbm, zeros_hbm, out_hbm, didx, ones_v,
                   stage, acc, sem0, sem1):
        c = lax.axis_index("c")
        s = lax.axis_index("s")
        pltpu.sync_copy(zeros_hbm, stage)

        @pl.loop(0, RPT // RB)
        def _(k):
            pltpu.sync_copy(stage, acc.at[pl.ds(s * RPT + k * RB, RB)])

        pltpu.sync_copy(ones_hbm, ones_v)
        b0 = (c * NS + s) * NB2
        plsc.subcore_barrier()

        def fire(i, sem):
            pltpu.async_copy(ones_v, acc.at[didx.at[i]], sem, add=True)

        def drain(i, sem):
            pltpu.make_async_copy(ones_v, acc.at[didx.at[i]], sem).wait()

        @pl.loop(0, NB2 // CH)
        def _(ci):
            pltpu.sync_copy(dst_hbm.at[pl.ds(b0 + ci * CH, CH)], didx)
            fire(0, sem0)
            fire(1, sem1)

            @pl.loop(0, CH // 2 - 1)
            def _(k):
                i0 = 2 * k
                drain(i0, sem0)
                fire(i0 + 2, sem0)
                drain(i0 + 1, sem1)
                fire(i0 + 3, sem1)

            drain(CH - 2, sem0)
            drain(CH - 1, sem1)

        plsc.subcore_barrier()

        @pl.loop(0, RPT // RB)
        def _(k):
            r0 = s * RPT + k * RB
            pltpu.sync_copy(acc.at[pl.ds(r0, RB)], stage)
            pltpu.sync_copy(stage, out_hbm.at[c].at[pl.ds(r0, RB)])

    return deg_kernel(dst, ones_blk, zeros_blk)


def _segment_sum(src, dst, table, zeros_blk, width):
    @functools.partial(
        pl.kernel,
        out_type=jax.ShapeDtypeStruct((NC, NP, width), jnp.float32),
        mesh=_MESH,
        scratch_types=[
            pltpu.VMEM((CH, EB), jnp.int32),
            pltpu.VMEM((CH, EB), jnp.int32),
            pltpu.VMEM((EB, width), jnp.float32),
            pltpu.VMEM((EB, width), jnp.float32),
            pltpu.VMEM_SHARED((NP, width), jnp.float32),
            pltpu.SemaphoreType.DMA,
            pltpu.SemaphoreType.DMA,
        ],
    )
    def seg_kernel(src_hbm, dst_hbm, tab_hbm, zeros_hbm, out_hbm,
                   sidx, didx, rows0, rows1, acc, gs0, gs1):
        c = lax.axis_index("c")
        s = lax.axis_index("s")
        pltpu.sync_copy(zeros_hbm, rows0)

        @pl.loop(0, RPT // RB)
        def _(k):
            pltpu.sync_copy(rows0, acc.at[pl.ds(s * RPT + k * RB, RB)])

        b0 = (c * NS + s) * NB2
        plsc.subcore_barrier()

        def fireg(i, rows, sem):
            pltpu.async_copy(tab_hbm.at[sidx.at[i]], rows, sem)

        def draing(rows, sem):
            pltpu.make_async_copy(tab_hbm.at[sidx.at[0]], rows, sem).wait()

        def scat(i, rows):
            pltpu.sync_copy(rows, acc.at[didx.at[i]], add=True)

        @pl.loop(0, NB2 // CH)
        def _(ci):
            pltpu.sync_copy(src_hbm.at[pl.ds(b0 + ci * CH, CH)], sidx)
            pltpu.sync_copy(dst_hbm.at[pl.ds(b0 + ci * CH, CH)], didx)
            fireg(0, rows0, gs0)
            fireg(1, rows1, gs1)

            @pl.loop(0, CH // 2 - 1)
            def _(k):
                i0 = 2 * k
                draing(rows0, gs0)
                scat(i0, rows0)
                fireg(i0 + 2, rows0, gs0)
                draing(rows1, gs1)
                scat(i0 + 1, rows1)
                fireg(i0 + 3, rows1, gs1)

            draing(rows0, gs0)
            scat(CH - 2, rows0)
            draing(rows1, gs1)
            scat(CH - 1, rows1)

        plsc.subcore_barrier()

        @pl.loop(0, RPT // RB)
        def _(k):
            r0 = s * RPT + k * RB
            pltpu.sync_copy(acc.at[pl.ds(r0, RB)], rows0)
            pltpu.sync_copy(rows0, out_hbm.at[c].at[pl.ds(r0, RB)])

    return seg_kernel(src, dst, table, zeros_blk)


BR = 1000


def _deg_specs():
    return [
        pl.BlockSpec((1, BR, 128), lambda i: (0, i, 0)),
        pl.BlockSpec((1, BR, 128), lambda i: (1, i, 0)),
    ]


def _dis_from(pa_ref, pb_ref):
    deg = 1.0 + pa_ref[0][:, :1] + pb_ref[0][:, :1]
    return lax.rsqrt(deg), 1.0 / deg


def _tc_matmul(x, W):
    d_out = W.shape[1]

    def body(x_ref, w_ref, h_ref):
        h_ref[...] = jnp.dot(x_ref[...], w_ref[...],
                             preferred_element_type=jnp.float32)

    return pl.pallas_call(
        body,
        grid=(N // BR,),
        in_specs=[
            pl.BlockSpec((BR, x.shape[1]), lambda i: (i, 0)),
            pl.BlockSpec(W.shape, lambda i: (0, 0)),
        ],
        out_specs=pl.BlockSpec((BR, d_out), lambda i: (i, 0)),
        out_shape=jax.ShapeDtypeStruct((N, d_out), jnp.float32),
    )(x, W)


def _tc_scale(H, degp):
    d_out = H.shape[1]

    def body(h_ref, pa_ref, pb_ref, hs_ref):
        dis, _ = _dis_from(pa_ref, pb_ref)
        hs_ref[...] = h_ref[...] * dis

    return pl.pallas_call(
        body,
        grid=(N // BR,),
        in_specs=[
            pl.BlockSpec((BR, d_out), lambda i: (i, 0)),
        ] + _deg_specs(),
        out_specs=pl.BlockSpec((BR, d_out), lambda i: (i, 0)),
        out_shape=jax.ShapeDtypeStruct((N, d_out), jnp.float32),
    )(H, degp, degp)


def _tc_mid(S, Hp, degp, b, g, beta, Wn):
    w_in = Hp.shape[1]
    d_out = Wn.shape[1]

    def body(sa_ref, sb_ref, hp_ref, pa_ref, pb_ref, b_ref, g_ref, beta_ref,
             w_ref, h_ref, hs_ref):
        dis, dis2 = _dis_from(pa_ref, pb_ref)
        z = (dis * (sa_ref[0][:, :w_in] + sb_ref[0][:, :w_in])
             + dis2 * hp_ref[...] + b_ref[...])
        scale = g_ref[...] / jnp.sqrt(1.0 + EPS)
        xact = jnp.maximum(z * scale + beta_ref[...], 0.0)
        h = jnp.dot(xact, w_ref[...], preferred_element_type=jnp.float32)
        h_ref[...] = h
        if d_out == 128:
            hs_ref[...] = h * dis
        else:
            hs_ref[...] = jnp.concatenate(
                [h * dis, jnp.zeros((h.shape[0], 128 - d_out), jnp.float32)],
                axis=1)

    return pl.pallas_call(
        body,
        grid=(N // BR,),
        in_specs=[
            pl.BlockSpec((1, BR, 128), lambda i: (0, i, 0)),
            pl.BlockSpec((1, BR, 128), lambda i: (1, i, 0)),
            pl.BlockSpec((BR, w_in), lambda i: (i, 0)),
        ] + _deg_specs() + [
            pl.BlockSpec((1, w_in), lambda i: (0, 0)),
            pl.BlockSpec((1, w_in), lambda i: (0, 0)),
            pl.BlockSpec((1, w_in), lambda i: (0, 0)),
            pl.BlockSpec(Wn.shape, lambda i: (0, 0)),
        ],
        out_specs=[
            pl.BlockSpec((BR, d_out), lambda i: (i, 0)),
            pl.BlockSpec((BR, 128), lambda i: (i, 0)),
        ],
        out_shape=[
            jax.ShapeDtypeStruct((N, d_out), jnp.float32),
            jax.ShapeDtypeStruct((N, 128), jnp.float32),
        ],
    )(S, S, Hp, degp, degp, b, g, beta, Wn)


def _tc_final(S, Hp, degp, b, g, beta):
    w_in = Hp.shape[1]

    def body(sa_ref, sb_ref, hp_ref, pa_ref, pb_ref, b_ref, g_ref, beta_ref,
             o_ref):
        dis, dis2 = _dis_from(pa_ref, pb_ref)
        z = (dis * (sa_ref[0][:, :w_in] + sb_ref[0][:, :w_in])
             + dis2 * hp_ref[...] + b_ref[...])
        scale = g_ref[...] / jnp.sqrt(1.0 + EPS)
        o_ref[...] = z * scale + beta_ref[...]

    return pl.pallas_call(
        body,
        grid=(N // BR,),
        in_specs=[
            pl.BlockSpec((1, BR, 128), lambda i: (0, i, 0)),
            pl.BlockSpec((1, BR, 128), lambda i: (1, i, 0)),
            pl.BlockSpec((BR, w_in), lambda i: (i, 0)),
        ] + _deg_specs() + [
            pl.BlockSpec((1, w_in), lambda i: (0, 0)),
            pl.BlockSpec((1, w_in), lambda i: (0, 0)),
            pl.BlockSpec((1, w_in), lambda i: (0, 0)),
        ],
        out_specs=pl.BlockSpec((BR, w_in), lambda i: (i, 0)),
        out_shape=jax.ShapeDtypeStruct((N, w_in), jnp.float32),
    )(S, S, Hp, degp, degp, b, g, beta)


def kernel(x, edge_index, W1, b1, W2, b2, W3, b3,
           g1, beta1, g2, beta2, g3, beta3):
    npad = E2 - E
    pad_src = (jnp.arange(npad, dtype=jnp.int32) * 37) % N
    pad_dst = N + (jnp.arange(npad, dtype=jnp.int32) % (NP - N))
    src = jnp.concatenate([edge_index[0], pad_src]).reshape(E2 // EB, EB)
    dst = jnp.concatenate([edge_index[1], pad_dst]).reshape(E2 // EB, EB)

    ones_blk = jnp.ones((EB, 128), jnp.float32)
    zeros128 = jnp.zeros((RB, 128), jnp.float32)

    degp = _deg_hist(dst, ones_blk, zeros128)

    b1r, b2r, b3r = b1.reshape(1, -1), b2.reshape(1, -1), b3.reshape(1, -1)
    g1r, g2r, g3r = g1.reshape(1, -1), g2.reshape(1, -1), g3.reshape(1, -1)
    be1r, be2r, be3r = (beta1.reshape(1, -1), beta2.reshape(1, -1),
                        beta3.reshape(1, -1))

    H1 = _tc_matmul(x, W1)
    Hs1 = _tc_scale(H1, degp)
    S1 = _segment_sum(src, dst, Hs1, zeros128, 128)
    H2, Hs2 = _tc_mid(S1, H1, degp, b1r, g1r, be1r, W2)
    S2 = _segment_sum(src, dst, Hs2, zeros128, 128)
    H3, Hs3 = _tc_mid(S2, H2, degp, b2r, g2r, be2r, W3)
    S3 = _segment_sum(src, dst, Hs3, zeros128, 128)
    out = _tc_final(S3, H3, degp, b3r, g3r, be3r)
    return out


# speedup vs baseline: 1.1548x; 1.0413x over previous
import functools

import jax
import jax.numpy as jnp
from jax import lax
from jax.experimental import pallas as pl
from jax.experimental.pallas import tpu as pltpu
from jax.experimental.pallas import tpu_sc as plsc

N = 10000
NP = 10240
E = 320000
EPS = 1e-5

NC = 2
NS = 16
EB = 128
E2 = 327680
NB2 = E2 // EB // (NC * NS)
CH = 16
RPT = NP // NS
RB = 128

_MESH = plsc.VectorSubcoreMesh(core_axis_name="c", subcore_axis_name="s")


def _deg_hist(dst, ones_blk, zeros_blk):
    @functools.partial(
        pl.kernel,
        out_type=jax.ShapeDtypeStruct((NC, NP, 128), jnp.float32),
        mesh=_MESH,
        scratch_types=[
            pltpu.VMEM((CH, EB), jnp.int32),
            pltpu.VMEM((EB, 128), jnp.float32),
            pltpu.VMEM((RB, 128), jnp.float32),
            pltpu.VMEM_SHARED((NP, 128), jnp.float32),
            pltpu.SemaphoreType.DMA,
            pltpu.SemaphoreType.DMA,
        ],
    )
    def deg_kernel(dst_hbm, ones_hbm, zeros_hbm, out_hbm, didx, ones_v,
                   stage, acc, sem0, sem1):
        c = lax.axis_index("c")
        s = lax.axis_index("s")
        pltpu.sync_copy(zeros_hbm, stage)

        @pl.loop(0, RPT // RB)
        def _(k):
            pltpu.sync_copy(stage, acc.at[pl.ds(s * RPT + k * RB, RB)])

        pltpu.sync_copy(ones_hbm, ones_v)
        b0 = (c * NS + s) * NB2
        plsc.subcore_barrier()

        def fire(i, sem):
            pltpu.async_copy(ones_v, acc.at[didx.at[i]], sem, add=True)

        def drain(i, sem):
            pltpu.make_async_copy(ones_v, acc.at[didx.at[i]], sem).wait()

        @pl.loop(0, NB2 // CH)
        def _(ci):
            pltpu.sync_copy(dst_hbm.at[pl.ds(b0 + ci * CH, CH)], didx)
            fire(0, sem0)
            fire(1, sem1)

            @pl.loop(0, CH // 2 - 1)
            def _(k):
                i0 = 2 * k
                drain(i0, sem0)
                fire(i0 + 2, sem0)
                drain(i0 + 1, sem1)
                fire(i0 + 3, sem1)

            drain(CH - 2, sem0)
            drain(CH - 1, sem1)

        plsc.subcore_barrier()

        @pl.loop(0, RPT // RB)
        def _(k):
            r0 = s * RPT + k * RB
            pltpu.sync_copy(acc.at[pl.ds(r0, RB)], stage)
            pltpu.sync_copy(stage, out_hbm.at[c].at[pl.ds(r0, RB)])

    return deg_kernel(dst, ones_blk, zeros_blk)


def _segment_sum(src, dst, table, zeros_blk, width):
    @functools.partial(
        pl.kernel,
        out_type=jax.ShapeDtypeStruct((NC, NP, width), jnp.float32),
        mesh=_MESH,
        scratch_types=[
            pltpu.VMEM((CH, EB), jnp.int32),
            pltpu.VMEM((CH, EB), jnp.int32),
            pltpu.VMEM((CH, EB), jnp.int32),
            pltpu.VMEM((CH, EB), jnp.int32),
            pltpu.VMEM((EB, width), jnp.float32),
            pltpu.VMEM((EB, width), jnp.float32),
            pltpu.VMEM_SHARED((NP, width), jnp.float32),
            pltpu.SemaphoreType.DMA,
            pltpu.SemaphoreType.DMA,
            pltpu.SemaphoreType.DMA,
            pltpu.SemaphoreType.DMA,
        ],
    )
    def seg_kernel(src_hbm, dst_hbm, tab_hbm, zeros_hbm, out_hbm,
                   sidxA, sidxB, didxA, didxB, rows0, rows1, acc,
                   gs0, gs1, isem, osem):
        c = lax.axis_index("c")
        s = lax.axis_index("s")
        b0 = (c * NS + s) * NB2
        nchunk = NB2 // CH
        nout = RPT // RB

        pltpu.sync_copy(zeros_hbm, rows0)
        for k in range(nout):
            pltpu.async_copy(rows0, acc.at[pl.ds(s * RPT + k * RB, RB)], osem)
        pltpu.async_copy(src_hbm.at[pl.ds(b0, CH)], sidxA, isem)
        pltpu.async_copy(dst_hbm.at[pl.ds(b0, CH)], didxA, isem)
        for k in range(nout):
            pltpu.make_async_copy(rows0, acc.at[pl.ds(s * RPT, RB)],
                                  osem).wait()
        pltpu.make_async_copy(src_hbm.at[pl.ds(b0, CH)], sidxA, isem).wait()
        pltpu.make_async_copy(dst_hbm.at[pl.ds(b0, CH)], didxA, isem).wait()
        plsc.subcore_barrier()

        def chunk(ci, sidx, didx, sidx_n, didx_n):
            if ci + 1 < nchunk:
                off = b0 + (ci + 1) * CH
                pltpu.async_copy(src_hbm.at[pl.ds(off, CH)], sidx_n, isem)
                pltpu.async_copy(dst_hbm.at[pl.ds(off, CH)], didx_n, isem)

            def fireg(i, rows, sem):
                pltpu.async_copy(tab_hbm.at[sidx.at[i]], rows, sem)

            def draing(rows, sem):
                pltpu.make_async_copy(tab_hbm.at[sidx.at[0]], rows, sem).wait()

            def scat(i, rows):
                pltpu.sync_copy(rows, acc.at[didx.at[i]], add=True)

            fireg(0, rows0, gs0)
            fireg(1, rows1, gs1)

            @pl.loop(0, CH // 2 - 1)
            def _(k):
                i0 = 2 * k
                draing(rows0, gs0)
                scat(i0, rows0)
                fireg(i0 + 2, rows0, gs0)
                draing(rows1, gs1)
                scat(i0 + 1, rows1)
                fireg(i0 + 3, rows1, gs1)

            draing(rows0, gs0)
            scat(CH - 2, rows0)
            draing(rows1, gs1)
            scat(CH - 1, rows1)

            if ci + 1 < nchunk:
                off = b0 + (ci + 1) * CH
                pltpu.make_async_copy(src_hbm.at[pl.ds(off, CH)], sidx_n,
                                      isem).wait()
                pltpu.make_async_copy(dst_hbm.at[pl.ds(off, CH)], didx_n,
                                      isem).wait()

        bufs = [(sidxA, didxA), (sidxB, didxB)]
        for ci in range(nchunk):
            chunk(ci, *bufs[ci % 2], *bufs[(ci + 1) % 2])

        plsc.subcore_barrier()

        rbufs = [rows0, rows1]

        def owrite(k, rows, fire):
            cp = (pltpu.async_copy if fire else pltpu.make_async_copy)
            r = cp(rows, out_hbm.at[c].at[pl.ds(s * RPT + k * RB, RB)], osem)
            if not fire:
                r.wait()

        for k in range(nout):
            if k >= 2:
                owrite(k - 2, rbufs[k % 2], False)
            pltpu.sync_copy(acc.at[pl.ds(s * RPT + k * RB, RB)], rbufs[k % 2])
            owrite(k, rbufs[k % 2], True)
        for k in range(nout - 2, nout):
            owrite(k, rbufs[k % 2], False)

    return seg_kernel(src, dst, table, zeros_blk)


BR = 1000


def _deg_specs():
    return [
        pl.BlockSpec((1, BR, 128), lambda i: (0, i, 0)),
        pl.BlockSpec((1, BR, 128), lambda i: (1, i, 0)),
    ]


def _dis_from(pa_ref, pb_ref):
    deg = 1.0 + pa_ref[0][:, :1] + pb_ref[0][:, :1]
    return lax.rsqrt(deg), 1.0 / deg


def _tc_matmul(x, W):
    d_out = W.shape[1]

    def body(x_ref, w_ref, h_ref):
        h_ref[...] = jnp.dot(x_ref[...], w_ref[...],
                             preferred_element_type=jnp.float32)

    return pl.pallas_call(
        body,
        grid=(N // BR,),
        in_specs=[
            pl.BlockSpec((BR, x.shape[1]), lambda i: (i, 0)),
            pl.BlockSpec(W.shape, lambda i: (0, 0)),
        ],
        out_specs=pl.BlockSpec((BR, d_out), lambda i: (i, 0)),
        out_shape=jax.ShapeDtypeStruct((N, d_out), jnp.float32),
    )(x, W)


def _tc_scale(H, degp):
    d_out = H.shape[1]

    def body(h_ref, pa_ref, pb_ref, hs_ref):
        dis, _ = _dis_from(pa_ref, pb_ref)
        hs_ref[...] = h_ref[...] * dis

    return pl.pallas_call(
        body,
        grid=(N // BR,),
        in_specs=[
            pl.BlockSpec((BR, d_out), lambda i: (i, 0)),
        ] + _deg_specs(),
        out_specs=pl.BlockSpec((BR, d_out), lambda i: (i, 0)),
        out_shape=jax.ShapeDtypeStruct((N, d_out), jnp.float32),
    )(H, degp, degp)


def _tc_mid(S, Hp, degp, b, g, beta, Wn):
    w_in = Hp.shape[1]
    d_out = Wn.shape[1]

    def body(sa_ref, sb_ref, hp_ref, pa_ref, pb_ref, b_ref, g_ref, beta_ref,
             w_ref, h_ref, hs_ref):
        dis, dis2 = _dis_from(pa_ref, pb_ref)
        z = (dis * (sa_ref[0][:, :w_in] + sb_ref[0][:, :w_in])
             + dis2 * hp_ref[...] + b_ref[...])
        scale = g_ref[...] / jnp.sqrt(1.0 + EPS)
        xact = jnp.maximum(z * scale + beta_ref[...], 0.0)
        h = jnp.dot(xact, w_ref[...], preferred_element_type=jnp.float32)
        h_ref[...] = h
        if d_out == 128:
            hs_ref[...] = h * dis
        else:
            hs_ref[...] = jnp.concatenate(
                [h * dis, jnp.zeros((h.shape[0], 128 - d_out), jnp.float32)],
                axis=1)

    return pl.pallas_call(
        body,
        grid=(N // BR,),
        in_specs=[
            pl.BlockSpec((1, BR, 128), lambda i: (0, i, 0)),
            pl.BlockSpec((1, BR, 128), lambda i: (1, i, 0)),
            pl.BlockSpec((BR, w_in), lambda i: (i, 0)),
        ] + _deg_specs() + [
            pl.BlockSpec((1, w_in), lambda i: (0, 0)),
            pl.BlockSpec((1, w_in), lambda i: (0, 0)),
            pl.BlockSpec((1, w_in), lambda i: (0, 0)),
            pl.BlockSpec(Wn.shape, lambda i: (0, 0)),
        ],
        out_specs=[
            pl.BlockSpec((BR, d_out), lambda i: (i, 0)),
            pl.BlockSpec((BR, 128), lambda i: (i, 0)),
        ],
        out_shape=[
            jax.ShapeDtypeStruct((N, d_out), jnp.float32),
            jax.ShapeDtypeStruct((N, 128), jnp.float32),
        ],
    )(S, S, Hp, degp, degp, b, g, beta, Wn)


def _tc_final(S, Hp, degp, b, g, beta):
    w_in = Hp.shape[1]

    def body(sa_ref, sb_ref, hp_ref, pa_ref, pb_ref, b_ref, g_ref, beta_ref,
             o_ref):
        dis, dis2 = _dis_from(pa_ref, pb_ref)
        z = (dis * (sa_ref[0][:, :w_in] + sb_ref[0][:, :w_in])
             + dis2 * hp_ref[...] + b_ref[...])
        scale = g_ref[...] / jnp.sqrt(1.0 + EPS)
        o_ref[...] = z * scale + beta_ref[...]

    return pl.pallas_call(
        body,
        grid=(N // BR,),
        in_specs=[
            pl.BlockSpec((1, BR, 128), lambda i: (0, i, 0)),
            pl.BlockSpec((1, BR, 128), lambda i: (1, i, 0)),
            pl.BlockSpec((BR, w_in), lambda i: (i, 0)),
        ] + _deg_specs() + [
            pl.BlockSpec((1, w_in), lambda i: (0, 0)),
            pl.BlockSpec((1, w_in), lambda i: (0, 0)),
            pl.BlockSpec((1, w_in), lambda i: (0, 0)),
        ],
        out_specs=pl.BlockSpec((BR, w_in), lambda i: (i, 0)),
        out_shape=jax.ShapeDtypeStruct((N, w_in), jnp.float32),
    )(S, S, Hp, degp, degp, b, g, beta)


def kernel(x, edge_index, W1, b1, W2, b2, W3, b3,
           g1, beta1, g2, beta2, g3, beta3):
    npad = E2 - E
    pad_src = (jnp.arange(npad, dtype=jnp.int32) * 37) % N
    pad_dst = N + (jnp.arange(npad, dtype=jnp.int32) % (NP - N))
    src = jnp.concatenate([edge_index[0], pad_src]).reshape(E2 // EB, EB)
    dst = jnp.concatenate([edge_index[1], pad_dst]).reshape(E2 // EB, EB)

    ones_blk = jnp.ones((EB, 128), jnp.float32)
    zeros128 = jnp.zeros((RB, 128), jnp.float32)

    degp = _deg_hist(dst, ones_blk, zeros128)

    b1r, b2r, b3r = b1.reshape(1, -1), b2.reshape(1, -1), b3.reshape(1, -1)
    g1r, g2r, g3r = g1.reshape(1, -1), g2.reshape(1, -1), g3.reshape(1, -1)
    be1r, be2r, be3r = (beta1.reshape(1, -1), beta2.reshape(1, -1),
                        beta3.reshape(1, -1))

    H1 = _tc_matmul(x, W1)
    Hs1 = _tc_scale(H1, degp)
    S1 = _segment_sum(src, dst, Hs1, zeros128, 128)
    H2, Hs2 = _tc_mid(S1, H1, degp, b1r, g1r, be1r, W2)
    S2 = _segment_sum(src, dst, Hs2, zeros128, 128)
    H3, Hs3 = _tc_mid(S2, H2, degp, b2r, g2r, be2r, W3)
    S3 = _segment_sum(src, dst, Hs3, zeros128, 128)
    out = _tc_final(S3, H3, degp, b3r, g3r, be3r)
    return out


# speedup vs baseline: 1.1740x; 1.0166x over previous
import functools

import jax
import jax.numpy as jnp
from jax import lax
from jax.experimental import pallas as pl
from jax.experimental.pallas import tpu as pltpu
from jax.experimental.pallas import tpu_sc as plsc

N = 10000
NP = 10240
E = 320000
EPS = 1e-5

NC = 2
NS = 16
EB = 128
E2 = 327680
NB2 = E2 // EB // (NC * NS)
CH = 16
RPT = NP // NS
RB = 128

_MESH = plsc.VectorSubcoreMesh(core_axis_name="c", subcore_axis_name="s")


def _deg_hist(dst, ones_blk, zeros_blk):
    @functools.partial(
        pl.kernel,
        out_type=jax.ShapeDtypeStruct((NC, NP, 128), jnp.float32),
        mesh=_MESH,
        scratch_types=[
            pltpu.VMEM((CH, EB), jnp.int32),
            pltpu.VMEM((CH, EB), jnp.int32),
            pltpu.VMEM((EB, 128), jnp.float32),
            pltpu.VMEM((RB, 128), jnp.float32),
            pltpu.VMEM_SHARED((NP, 128), jnp.float32),
            pltpu.SemaphoreType.DMA,
            pltpu.SemaphoreType.DMA,
            pltpu.SemaphoreType.DMA,
            pltpu.SemaphoreType.DMA,
        ],
    )
    def deg_kernel(dst_hbm, ones_hbm, zeros_hbm, out_hbm, didxA, didxB,
                   ones_v, stage, acc, sem0, sem1, isem, osem):
        c = lax.axis_index("c")
        s = lax.axis_index("s")
        b0 = (c * NS + s) * NB2
        nchunk = NB2 // CH
        nout = RPT // RB

        pltpu.sync_copy(zeros_hbm, stage)
        for k in range(nout):
            pltpu.async_copy(stage, acc.at[pl.ds(s * RPT + k * RB, RB)], osem)
        pltpu.async_copy(ones_hbm, ones_v, isem)
        pltpu.async_copy(dst_hbm.at[pl.ds(b0, CH)], didxA, isem)
        for k in range(nout):
            pltpu.make_async_copy(stage, acc.at[pl.ds(s * RPT, RB)],
                                  osem).wait()
        pltpu.make_async_copy(ones_hbm, ones_v, isem).wait()
        pltpu.make_async_copy(dst_hbm.at[pl.ds(b0, CH)], didxA, isem).wait()
        plsc.subcore_barrier()

        def chunk(ci, didx, didx_n):
            if ci + 1 < nchunk:
                off = b0 + (ci + 1) * CH
                pltpu.async_copy(dst_hbm.at[pl.ds(off, CH)], didx_n, isem)

            def fire(i, sem):
                pltpu.async_copy(ones_v, acc.at[didx.at[i]], sem, add=True)

            def drain(i, sem):
                pltpu.make_async_copy(ones_v, acc.at[didx.at[i]], sem).wait()

            fire(0, sem0)
            fire(1, sem1)

            @pl.loop(0, CH // 2 - 1)
            def _(k):
                i0 = 2 * k
                drain(i0, sem0)
                fire(i0 + 2, sem0)
                drain(i0 + 1, sem1)
                fire(i0 + 3, sem1)

            drain(CH - 2, sem0)
            drain(CH - 1, sem1)

            if ci + 1 < nchunk:
                off = b0 + (ci + 1) * CH
                pltpu.make_async_copy(dst_hbm.at[pl.ds(off, CH)], didx_n,
                                      isem).wait()

        bufs = [didxA, didxB]
        for ci in range(nchunk):
            chunk(ci, bufs[ci % 2], bufs[(ci + 1) % 2])

        plsc.subcore_barrier()

        rbufs = [stage, ones_v]

        def owrite(k, rows, fire):
            cp = (pltpu.async_copy if fire else pltpu.make_async_copy)
            r = cp(rows, out_hbm.at[c].at[pl.ds(s * RPT + k * RB, RB)], osem)
            if not fire:
                r.wait()

        for k in range(nout):
            if k >= 2:
                owrite(k - 2, rbufs[k % 2], False)
            pltpu.sync_copy(acc.at[pl.ds(s * RPT + k * RB, RB)], rbufs[k % 2])
            owrite(k, rbufs[k % 2], True)
        for k in range(nout - 2, nout):
            owrite(k, rbufs[k % 2], False)

    return deg_kernel(dst, ones_blk, zeros_blk)


def _segment_sum(src, dst, table, zeros_blk, width):
    @functools.partial(
        pl.kernel,
        out_type=jax.ShapeDtypeStruct((NC, NP, width), jnp.float32),
        mesh=_MESH,
        scratch_types=[
            pltpu.VMEM((CH, EB), jnp.int32),
            pltpu.VMEM((CH, EB), jnp.int32),
            pltpu.VMEM((CH, EB), jnp.int32),
            pltpu.VMEM((CH, EB), jnp.int32),
            pltpu.VMEM((EB, width), jnp.float32),
            pltpu.VMEM((EB, width), jnp.float32),
            pltpu.VMEM_SHARED((NP, width), jnp.float32),
            pltpu.SemaphoreType.DMA,
            pltpu.SemaphoreType.DMA,
            pltpu.SemaphoreType.DMA,
            pltpu.SemaphoreType.DMA,
        ],
    )
    def seg_kernel(src_hbm, dst_hbm, tab_hbm, zeros_hbm, out_hbm,
                   sidxA, sidxB, didxA, didxB, rows0, rows1, acc,
                   gs0, gs1, isem, osem):
        c = lax.axis_index("c")
        s = lax.axis_index("s")
        b0 = (c * NS + s) * NB2
        nchunk = NB2 // CH
        nout = RPT // RB

        pltpu.sync_copy(zeros_hbm, rows0)
        for k in range(nout):
            pltpu.async_copy(rows0, acc.at[pl.ds(s * RPT + k * RB, RB)], osem)
        pltpu.async_copy(src_hbm.at[pl.ds(b0, CH)], sidxA, isem)
        pltpu.async_copy(dst_hbm.at[pl.ds(b0, CH)], didxA, isem)
        for k in range(nout):
            pltpu.make_async_copy(rows0, acc.at[pl.ds(s * RPT, RB)],
                                  osem).wait()
        pltpu.make_async_copy(src_hbm.at[pl.ds(b0, CH)], sidxA, isem).wait()
        pltpu.make_async_copy(dst_hbm.at[pl.ds(b0, CH)], didxA, isem).wait()
        plsc.subcore_barrier()

        def chunk(ci, sidx, didx, sidx_n, didx_n):
            if ci + 1 < nchunk:
                off = b0 + (ci + 1) * CH
                pltpu.async_copy(src_hbm.at[pl.ds(off, CH)], sidx_n, isem)
                pltpu.async_copy(dst_hbm.at[pl.ds(off, CH)], didx_n, isem)

            def fireg(i, rows, sem):
                pltpu.async_copy(tab_hbm.at[sidx.at[i]], rows, sem)

            def draing(rows, sem):
                pltpu.make_async_copy(tab_hbm.at[sidx.at[0]], rows, sem).wait()

            def scat(i, rows):
                pltpu.sync_copy(rows, acc.at[didx.at[i]], add=True)

            fireg(0, rows0, gs0)
            fireg(1, rows1, gs1)

            @pl.loop(0, CH // 2 - 1)
            def _(k):
                i0 = 2 * k
                draing(rows0, gs0)
                scat(i0, rows0)
                fireg(i0 + 2, rows0, gs0)
                draing(rows1, gs1)
                scat(i0 + 1, rows1)
                fireg(i0 + 3, rows1, gs1)

            draing(rows0, gs0)
            scat(CH - 2, rows0)
            draing(rows1, gs1)
            scat(CH - 1, rows1)

            if ci + 1 < nchunk:
                off = b0 + (ci + 1) * CH
                pltpu.make_async_copy(src_hbm.at[pl.ds(off, CH)], sidx_n,
                                      isem).wait()
                pltpu.make_async_copy(dst_hbm.at[pl.ds(off, CH)], didx_n,
                                      isem).wait()

        bufs = [(sidxA, didxA), (sidxB, didxB)]
        for ci in range(nchunk):
            chunk(ci, *bufs[ci % 2], *bufs[(ci + 1) % 2])

        plsc.subcore_barrier()

        rbufs = [rows0, rows1]

        def owrite(k, rows, fire):
            cp = (pltpu.async_copy if fire else pltpu.make_async_copy)
            r = cp(rows, out_hbm.at[c].at[pl.ds(s * RPT + k * RB, RB)], osem)
            if not fire:
                r.wait()

        for k in range(nout):
            if k >= 2:
                owrite(k - 2, rbufs[k % 2], False)
            pltpu.sync_copy(acc.at[pl.ds(s * RPT + k * RB, RB)], rbufs[k % 2])
            owrite(k, rbufs[k % 2], True)
        for k in range(nout - 2, nout):
            owrite(k, rbufs[k % 2], False)

    return seg_kernel(src, dst, table, zeros_blk)


BR = 1000


def _deg_specs():
    return [
        pl.BlockSpec((1, BR, 128), lambda i: (0, i, 0)),
        pl.BlockSpec((1, BR, 128), lambda i: (1, i, 0)),
    ]


def _dis_from(pa_ref, pb_ref):
    deg = 1.0 + pa_ref[0][:, :1] + pb_ref[0][:, :1]
    return lax.rsqrt(deg), 1.0 / deg


def _tc_matmul(x, W):
    d_out = W.shape[1]

    def body(x_ref, w_ref, h_ref):
        h_ref[...] = jnp.dot(x_ref[...], w_ref[...],
                             preferred_element_type=jnp.float32)

    return pl.pallas_call(
        body,
        grid=(N // BR,),
        in_specs=[
            pl.BlockSpec((BR, x.shape[1]), lambda i: (i, 0)),
            pl.BlockSpec(W.shape, lambda i: (0, 0)),
        ],
        out_specs=pl.BlockSpec((BR, d_out), lambda i: (i, 0)),
        out_shape=jax.ShapeDtypeStruct((N, d_out), jnp.float32),
    )(x, W)


def _tc_scale(H, degp):
    d_out = H.shape[1]

    def body(h_ref, pa_ref, pb_ref, hs_ref):
        dis, _ = _dis_from(pa_ref, pb_ref)
        hs_ref[...] = h_ref[...] * dis

    return pl.pallas_call(
        body,
        grid=(N // BR,),
        in_specs=[
            pl.BlockSpec((BR, d_out), lambda i: (i, 0)),
        ] + _deg_specs(),
        out_specs=pl.BlockSpec((BR, d_out), lambda i: (i, 0)),
        out_shape=jax.ShapeDtypeStruct((N, d_out), jnp.float32),
    )(H, degp, degp)


def _tc_mid(S, Hp, degp, b, g, beta, Wn):
    w_in = Hp.shape[1]
    d_out = Wn.shape[1]

    def body(sa_ref, sb_ref, hp_ref, pa_ref, pb_ref, b_ref, g_ref, beta_ref,
             w_ref, h_ref, hs_ref):
        dis, dis2 = _dis_from(pa_ref, pb_ref)
        z = (dis * (sa_ref[0][:, :w_in] + sb_ref[0][:, :w_in])
             + dis2 * hp_ref[...] + b_ref[...])
        scale = g_ref[...] / jnp.sqrt(1.0 + EPS)
        xact = jnp.maximum(z * scale + beta_ref[...], 0.0)
        h = jnp.dot(xact, w_ref[...], preferred_element_type=jnp.float32)
        h_ref[...] = h
        if d_out == 128:
            hs_ref[...] = h * dis
        else:
            hs_ref[...] = jnp.concatenate(
                [h * dis, jnp.zeros((h.shape[0], 128 - d_out), jnp.float32)],
                axis=1)

    return pl.pallas_call(
        body,
        grid=(N // BR,),
        in_specs=[
            pl.BlockSpec((1, BR, 128), lambda i: (0, i, 0)),
            pl.BlockSpec((1, BR, 128), lambda i: (1, i, 0)),
            pl.BlockSpec((BR, w_in), lambda i: (i, 0)),
        ] + _deg_specs() + [
            pl.BlockSpec((1, w_in), lambda i: (0, 0)),
            pl.BlockSpec((1, w_in), lambda i: (0, 0)),
            pl.BlockSpec((1, w_in), lambda i: (0, 0)),
            pl.BlockSpec(Wn.shape, lambda i: (0, 0)),
        ],
        out_specs=[
            pl.BlockSpec((BR, d_out), lambda i: (i, 0)),
            pl.BlockSpec((BR, 128), lambda i: (i, 0)),
        ],
        out_shape=[
            jax.ShapeDtypeStruct((N, d_out), jnp.float32),
            jax.ShapeDtypeStruct((N, 128), jnp.float32),
        ],
    )(S, S, Hp, degp, degp, b, g, beta, Wn)


def _tc_final(S, Hp, degp, b, g, beta):
    w_in = Hp.shape[1]

    def body(sa_ref, sb_ref, hp_ref, pa_ref, pb_ref, b_ref, g_ref, beta_ref,
             o_ref):
        dis, dis2 = _dis_from(pa_ref, pb_ref)
        z = (dis * (sa_ref[0][:, :w_in] + sb_ref[0][:, :w_in])
             + dis2 * hp_ref[...] + b_ref[...])
        scale = g_ref[...] / jnp.sqrt(1.0 + EPS)
        o_ref[...] = z * scale + beta_ref[...]

    return pl.pallas_call(
        body,
        grid=(N // BR,),
        in_specs=[
            pl.BlockSpec((1, BR, 128), lambda i: (0, i, 0)),
            pl.BlockSpec((1, BR, 128), lambda i: (1, i, 0)),
            pl.BlockSpec((BR, w_in), lambda i: (i, 0)),
        ] + _deg_specs() + [
            pl.BlockSpec((1, w_in), lambda i: (0, 0)),
            pl.BlockSpec((1, w_in), lambda i: (0, 0)),
            pl.BlockSpec((1, w_in), lambda i: (0, 0)),
        ],
        out_specs=pl.BlockSpec((BR, w_in), lambda i: (i, 0)),
        out_shape=jax.ShapeDtypeStruct((N, w_in), jnp.float32),
    )(S, S, Hp, degp, degp, b, g, beta)


def kernel(x, edge_index, W1, b1, W2, b2, W3, b3,
           g1, beta1, g2, beta2, g3, beta3):
    npad = E2 - E
    pad_src = (jnp.arange(npad, dtype=jnp.int32) * 37) % N
    pad_dst = N + (jnp.arange(npad, dtype=jnp.int32) % (NP - N))
    src = jnp.concatenate([edge_index[0], pad_src]).reshape(E2 // EB, EB)
    dst = jnp.concatenate([edge_index[1], pad_dst]).reshape(E2 // EB, EB)

    ones_blk = jnp.ones((EB, 128), jnp.float32)
    zeros128 = jnp.zeros((RB, 128), jnp.float32)

    degp = _deg_hist(dst, ones_blk, zeros128)

    b1r, b2r, b3r = b1.reshape(1, -1), b2.reshape(1, -1), b3.reshape(1, -1)
    g1r, g2r, g3r = g1.reshape(1, -1), g2.reshape(1, -1), g3.reshape(1, -1)
    be1r, be2r, be3r = (beta1.reshape(1, -1), beta2.reshape(1, -1),
                        beta3.reshape(1, -1))

    H1 = _tc_matmul(x, W1)
    Hs1 = _tc_scale(H1, degp)
    S1 = _segment_sum(src, dst, Hs1, zeros128, 128)
    H2, Hs2 = _tc_mid(S1, H1, degp, b1r, g1r, be1r, W2)
    S2 = _segment_sum(src, dst, Hs2, zeros128, 128)
    H3, Hs3 = _tc_mid(S2, H2, degp, b2r, g2r, be2r, W3)
    S3 = _segment_sum(src, dst, Hs3, zeros128, 128)
    out = _tc_final(S3, H3, degp, b3r, g3r, be3r)
    return out
